# baseline (speedup 1.0000x reference)
import functools

import jax
import jax.numpy as jnp
from jax import lax
from jax.experimental import pallas as pl
from jax.experimental.pallas import tpu as pltpu
from jax.experimental.pallas import tpu_sc as plsc

VOCAB = 1000000
D = 32
B = 4096
HIST = 20
C = 1000

_info = plsc.get_sparse_core_info()
_NC, _NS, _L = _info.num_cores, _info.num_subcores, _info.num_lanes
NW = _NC * _NS
B_PER_W = B // NW
IDX_PER_W = B_PER_W * HIST
CH_ELEMS = 16
CH_ROWS = CH_ELEMS * HIST
NCHUNK = B_PER_W // CH_ELEMS
TB = 4096
NTB = (VOCAB + TB - 1) // TB
QROWS = NTB * (TB // 4)


def _tc_relayout(tableT):
    def body(in_ref, out_ref):
        x = in_ref[...]
        eye = jnp.float32(
            lax.broadcasted_iota(jnp.int32, (D, D), 0)
            == lax.broadcasted_iota(jnp.int32, (D, D), 1))
        for s in range(4):
            out_ref[:, s * D:(s + 1) * D] = lax.dot_general(
                x[:, s * 1024:(s + 1) * 1024], eye,
                (((0,), (0,)), ((), ())),
                preferred_element_type=jnp.float32)

    return pl.pallas_call(
        body,
        grid=(NTB,),
        in_specs=[pl.BlockSpec((D, TB), lambda j: (0, j))],
        out_specs=pl.BlockSpec((TB // 4, 4 * D), lambda j: (j, 0)),
        out_shape=jax.ShapeDtypeStruct((QROWS, 4 * D), jnp.float32),
    )(tableT)


def _sc_bag(qidx, off, table4):
    mesh = plsc.VectorSubcoreMesh(core_axis_name="c", subcore_axis_name="s")

    @functools.partial(
        pl.kernel,
        mesh=mesh,
        out_type=jax.ShapeDtypeStruct((B // 4, 128), jnp.float32),
        scratch_types=[
            pltpu.VMEM((IDX_PER_W,), jnp.int32),
            pltpu.VMEM((IDX_PER_W,), jnp.int32),
            pltpu.VMEM((CH_ROWS, 128), jnp.float32),
            pltpu.VMEM((B_PER_W // 4, 128), jnp.float32),
            pltpu.SemaphoreType.DMA,
        ],
        compiler_params=pltpu.CompilerParams(needs_layout_passes=False),
    )
    def k(qidx_ref, off_ref, table_ref, bag_ref, qidx_v, off_v, buf, bag_v, sem):
        iota = lax.iota(jnp.int32, 16)
        wid = lax.axis_index("s") * _NC + lax.axis_index("c")
        base = wid * IDX_PER_W
        pltpu.sync_copy(qidx_ref.at[pl.ds(base, IDX_PER_W)], qidx_v)
        pltpu.sync_copy(off_ref.at[pl.ds(base, IDX_PER_W)], off_v)

        def chunk_body(c, carry):
            cbase = c * CH_ROWS
            cps = []
            for s, n in ((0, 128), (128, 128), (256, 64)):
                cps.append(pltpu.async_copy(
                    table_ref.at[qidx_v.at[pl.ds(cbase + s, n)]],
                    buf.at[pl.ds(s, n)],
                    sem))
            for cp in cps:
                cp.wait()
            for pair in range(CH_ELEMS // 2):
                for j in range(2):
                    e_lane = ((pair * 2 + j) % 4) * 32
                    e_row = c * (CH_ELEMS // 4) + pair // 2
                    for l in range(HIST):
                        r = (pair * 2 + j) * HIST + l
                        off_splat = plsc.load_gather(
                            off_v, [jnp.full((16,), cbase + r, jnp.int32)])
                        rowvec = jnp.full((16,), r, jnp.int32)
                        lanes0 = off_splat + iota
                        v0 = plsc.load_gather(buf, [rowvec, lanes0])
                        v1 = plsc.load_gather(buf, [rowvec, lanes0 + 16])
                        if l == 0:
                            acc0, acc1 = v0, v1
                        else:
                            acc0 = acc0 + v0
                            acc1 = acc1 + v1
                    bag_v[e_row, pl.ds(e_lane, 16)] = acc0
                    bag_v[e_row, pl.ds(e_lane + 16, 16)] = acc1
            return carry

        lax.fori_loop(0, NCHUNK, chunk_body, 0)
        pltpu.sync_copy(
            bag_v, bag_ref.at[pl.ds(wid * (B_PER_W // 4), B_PER_W // 4)])

    return k(qidx, off, table4)


def _tc_project_t(bag, W, b_scaled):
    BN = 1024

    def mm(w_ref, bag_ref, b_ref, out_ref):
        acc = lax.dot_general(
            w_ref[...], bag_ref[...],
            (((1,), (1,)), ((), ())),
            preferred_element_type=jnp.float32)
        out_ref[...] = acc + b_ref[...]

    return pl.pallas_call(
        mm,
        grid=(B // BN,),
        in_specs=[
            pl.BlockSpec((C, D), lambda j: (0, 0)),
            pl.BlockSpec((BN, D), lambda j: (j, 0)),
            pl.BlockSpec((C, 1), lambda j: (0, 0)),
        ],
        out_specs=pl.BlockSpec((C, BN), lambda j: (0, j)),
        out_shape=jax.ShapeDtypeStruct((C, B), jnp.float32),
    )(W, bag, b_scaled)


def kernel(inputs, embed_table, W, b):
    idx_flat = inputs.reshape(-1).astype(jnp.int32)
    qidx = ((idx_flat >> 12) << 10) | (idx_flat & 1023)
    off = ((idx_flat >> 10) & 3) << 5
    table4 = _tc_relayout(embed_table.T)
    bag = _sc_bag(qidx, off, table4).reshape(B, D)
    b_scaled = (b * jnp.float32(HIST)).reshape(C, 1)
    return _tc_project_t(bag, W, b_scaled).T


# speedup vs baseline: 1.7908x; 1.7908x over previous
import functools

import jax
import jax.numpy as jnp
from jax import lax
from jax.experimental import pallas as pl
from jax.experimental.pallas import tpu as pltpu
from jax.experimental.pallas import tpu_sc as plsc

VOCAB = 1000000
D = 32
B = 4096
HIST = 20
C = 1000

_info = plsc.get_sparse_core_info()
_NC, _NS, _L = _info.num_cores, _info.num_subcores, _info.num_lanes
NW = _NC * _NS
B_PER_W = B // NW
IDX_PER_W = B_PER_W * HIST
CH_ELEMS = 16
CH_ROWS = CH_ELEMS * HIST
NCHUNK = B_PER_W // CH_ELEMS
TB = 8192
NTB = (VOCAB + TB - 1) // TB
QROWS = NTB * (TB // 4)


def _tc_relayout(tableT):
    def body(in_ref, out_ref):
        x = in_ref[...]
        for k in range(TB // 4096):
            z = jnp.concatenate(
                [x[:, (4 * k + s) * 1024:(4 * k + s + 1) * 1024]
                 for s in range(4)], axis=0)
            out_ref[k * 1024:(k + 1) * 1024, :] = z.T

    return pl.pallas_call(
        body,
        grid=(NTB,),
        in_specs=[pl.BlockSpec((D, TB), lambda j: (0, j))],
        out_specs=pl.BlockSpec((TB // 4, 4 * D), lambda j: (j, 0)),
        out_shape=jax.ShapeDtypeStruct((QROWS, 4 * D), jnp.float32),
    )(tableT)


def _sc_bag(qidx, off, table4):
    mesh = plsc.VectorSubcoreMesh(core_axis_name="c", subcore_axis_name="s")

    @functools.partial(
        pl.kernel,
        mesh=mesh,
        out_type=jax.ShapeDtypeStruct((B // 4, 128), jnp.float32),
        scratch_types=[
            pltpu.VMEM((IDX_PER_W,), jnp.int32),
            pltpu.VMEM((IDX_PER_W,), jnp.int32),
            pltpu.VMEM((CH_ROWS, 128), jnp.float32),
            pltpu.VMEM((B_PER_W // 4, 128), jnp.float32),
            pltpu.SemaphoreType.DMA,
        ],
        compiler_params=pltpu.CompilerParams(needs_layout_passes=False),
    )
    def k(qidx_ref, off_ref, table_ref, bag_ref, qidx_v, off_v, buf, bag_v, sem):
        iota = lax.iota(jnp.int32, 16)
        wid = lax.axis_index("s") * _NC + lax.axis_index("c")
        base = wid * IDX_PER_W
        pltpu.sync_copy(qidx_ref.at[pl.ds(base, IDX_PER_W)], qidx_v)
        pltpu.sync_copy(off_ref.at[pl.ds(base, IDX_PER_W)], off_v)

        def chunk_body(c, carry):
            cbase = c * CH_ROWS
            cps = []
            for s, n in ((0, 128), (128, 128), (256, 64)):
                cps.append(pltpu.async_copy(
                    table_ref.at[qidx_v.at[pl.ds(cbase + s, n)]],
                    buf.at[pl.ds(s, n)],
                    sem))
            for cp in cps:
                cp.wait()
            for pair in range(CH_ELEMS // 2):
                for j in range(2):
                    e_lane = ((pair * 2 + j) % 4) * 32
                    e_row = c * (CH_ELEMS // 4) + pair // 2
                    for l in range(HIST):
                        r = (pair * 2 + j) * HIST + l
                        off_splat = plsc.load_gather(
                            off_v, [jnp.full((16,), cbase + r, jnp.int32)])
                        rowvec = jnp.full((16,), r, jnp.int32)
                        lanes0 = off_splat + iota
                        v0 = plsc.load_gather(buf, [rowvec, lanes0])
                        v1 = plsc.load_gather(buf, [rowvec, lanes0 + 16])
                        if l == 0:
                            acc0, acc1 = v0, v1
                        else:
                            acc0 = acc0 + v0
                            acc1 = acc1 + v1
                    bag_v[e_row, pl.ds(e_lane, 16)] = acc0
                    bag_v[e_row, pl.ds(e_lane + 16, 16)] = acc1
            return carry

        lax.fori_loop(0, NCHUNK, chunk_body, 0)
        pltpu.sync_copy(
            bag_v, bag_ref.at[pl.ds(wid * (B_PER_W // 4), B_PER_W // 4)])

    return k(qidx, off, table4)


def _tc_project_t(bag, W, b_scaled):
    BN = 1024

    def mm(w_ref, bag_ref, b_ref, out_ref):
        acc = lax.dot_general(
            w_ref[...], bag_ref[...],
            (((1,), (1,)), ((), ())),
            preferred_element_type=jnp.float32)
        out_ref[...] = acc + b_ref[...]

    return pl.pallas_call(
        mm,
        grid=(B // BN,),
        in_specs=[
            pl.BlockSpec((C, D), lambda j: (0, 0)),
            pl.BlockSpec((BN, D), lambda j: (j, 0)),
            pl.BlockSpec((C, 1), lambda j: (0, 0)),
        ],
        out_specs=pl.BlockSpec((C, BN), lambda j: (0, j)),
        out_shape=jax.ShapeDtypeStruct((C, B), jnp.float32),
    )(W, bag, b_scaled)


def kernel(inputs, embed_table, W, b):
    idx_flat = inputs.reshape(-1).astype(jnp.int32)
    qidx = ((idx_flat >> 12) << 10) | (idx_flat & 1023)
    off = ((idx_flat >> 10) & 3) << 5
    table4 = _tc_relayout(embed_table.T)
    bag = _sc_bag(qidx, off, table4).reshape(B, D)
    b_scaled = (b * jnp.float32(HIST)).reshape(C, 1)
    return _tc_project_t(bag, W, b_scaled).T


# speedup vs baseline: 2.1853x; 1.2203x over previous
import functools

import jax
import jax.numpy as jnp
from jax import lax
from jax.experimental import pallas as pl
from jax.experimental.pallas import tpu as pltpu
from jax.experimental.pallas import tpu_sc as plsc

VOCAB = 1000000
D = 32
B = 4096
HIST = 20
C = 1000

_info = plsc.get_sparse_core_info()
_NC, _NS, _L = _info.num_cores, _info.num_subcores, _info.num_lanes
NW = _NC * _NS
B_PER_W = B // NW
IDX_PER_W = B_PER_W * HIST
CH_ELEMS = 16
CH_ROWS = CH_ELEMS * HIST
NCHUNK = B_PER_W // CH_ELEMS
TB = 16384
NTB = (VOCAB + TB - 1) // TB
QROWS = NTB * (TB // 4)


def _tc_relayout(tableT):
    def body(in_ref, out_ref):
        x = in_ref[...]
        for k in range(TB // 4096):
            z = jnp.concatenate(
                [x[:, (4 * k + s) * 1024:(4 * k + s + 1) * 1024]
                 for s in range(4)], axis=0)
            out_ref[k * 1024:(k + 1) * 1024, :] = z.T

    return pl.pallas_call(
        body,
        grid=(NTB,),
        in_specs=[pl.BlockSpec((D, TB), lambda j: (0, j))],
        out_specs=pl.BlockSpec((TB // 4, 4 * D), lambda j: (j, 0)),
        out_shape=jax.ShapeDtypeStruct((QROWS, 4 * D), jnp.float32),
    )(tableT)


def _sc_bag(qidx, off, table4):
    mesh = plsc.VectorSubcoreMesh(core_axis_name="c", subcore_axis_name="s")

    @functools.partial(
        pl.kernel,
        mesh=mesh,
        out_type=jax.ShapeDtypeStruct((B // 4, 128), jnp.float32),
        scratch_types=[
            pltpu.VMEM((IDX_PER_W,), jnp.int32),
            pltpu.VMEM((IDX_PER_W,), jnp.int32),
            pltpu.VMEM((CH_ROWS, 128), jnp.float32),
            pltpu.VMEM((B_PER_W // 4, 128), jnp.float32),
            pltpu.SemaphoreType.DMA,
        ],
        compiler_params=pltpu.CompilerParams(needs_layout_passes=False),
    )
    def k(qidx_ref, off_ref, table_ref, bag_ref, qidx_v, off_v, buf, bag_v, sem):
        iota = lax.iota(jnp.int32, 16)
        wid = lax.axis_index("s") * _NC + lax.axis_index("c")
        base = wid * IDX_PER_W
        pltpu.sync_copy(qidx_ref.at[pl.ds(base, IDX_PER_W)], qidx_v)
        pltpu.sync_copy(off_ref.at[pl.ds(base, IDX_PER_W)], off_v)

        def chunk_body(c, carry):
            cbase = c * CH_ROWS
            cps = []
            for s, n in ((0, 128), (128, 128), (256, 64)):
                cps.append(pltpu.async_copy(
                    table_ref.at[qidx_v.at[pl.ds(cbase + s, n)]],
                    buf.at[pl.ds(s, n)],
                    sem))
            for cp in cps:
                cp.wait()
            for pair in range(CH_ELEMS // 2):
                for j in range(2):
                    e_lane = ((pair * 2 + j) % 4) * 32
                    e_row = c * (CH_ELEMS // 4) + pair // 2
                    for l in range(HIST):
                        r = (pair * 2 + j) * HIST + l
                        off_splat = plsc.load_gather(
                            off_v, [jnp.full((16,), cbase + r, jnp.int32)])
                        rowvec = jnp.full((16,), r, jnp.int32)
                        lanes0 = off_splat + iota
                        v0 = plsc.load_gather(buf, [rowvec, lanes0])
                        v1 = plsc.load_gather(buf, [rowvec, lanes0 + 16])
                        if l == 0:
                            acc0, acc1 = v0, v1
                        else:
                            acc0 = acc0 + v0
                            acc1 = acc1 + v1
                    bag_v[e_row, pl.ds(e_lane, 16)] = acc0
                    bag_v[e_row, pl.ds(e_lane + 16, 16)] = acc1
            return carry

        lax.fori_loop(0, NCHUNK, chunk_body, 0)
        pltpu.sync_copy(
            bag_v, bag_ref.at[pl.ds(wid * (B_PER_W // 4), B_PER_W // 4)])

    return k(qidx, off, table4)


def _tc_project_t(bag, W, b_scaled):
    BN = 1024

    def mm(w_ref, bag_ref, b_ref, out_ref):
        acc = lax.dot_general(
            w_ref[...], bag_ref[...],
            (((1,), (1,)), ((), ())),
            preferred_element_type=jnp.float32)
        out_ref[...] = acc + b_ref[...]

    return pl.pallas_call(
        mm,
        grid=(B // BN,),
        in_specs=[
            pl.BlockSpec((C, D), lambda j: (0, 0)),
            pl.BlockSpec((BN, D), lambda j: (j, 0)),
            pl.BlockSpec((C, 1), lambda j: (0, 0)),
        ],
        out_specs=pl.BlockSpec((C, BN), lambda j: (0, j)),
        out_shape=jax.ShapeDtypeStruct((C, B), jnp.float32),
    )(W, bag, b_scaled)


def kernel(inputs, embed_table, W, b):
    idx_flat = inputs.reshape(-1).astype(jnp.int32)
    qidx = ((idx_flat >> 12) << 10) | (idx_flat & 1023)
    off = ((idx_flat >> 10) & 3) << 5
    table4 = _tc_relayout(embed_table.T)
    bag = _sc_bag(qidx, off, table4).reshape(B, D)
    b_scaled = (b * jnp.float32(HIST)).reshape(C, 1)
    return _tc_project_t(bag, W, b_scaled).T


# speedup vs baseline: 2.3989x; 1.0977x over previous
import functools

import jax
import jax.numpy as jnp
from jax import lax
from jax.experimental import pallas as pl
from jax.experimental.pallas import tpu as pltpu
from jax.experimental.pallas import tpu_sc as plsc

VOCAB = 1000000
D = 32
B = 4096
HIST = 20
C = 1000

_info = plsc.get_sparse_core_info()
_NC, _NS, _L = _info.num_cores, _info.num_subcores, _info.num_lanes
NW = _NC * _NS
B_PER_W = B // NW
IDX_PER_W = B_PER_W * HIST
CH_ELEMS = 16
CH_ROWS = CH_ELEMS * HIST
NCHUNK = B_PER_W // CH_ELEMS
TB = 32768
NTB = (VOCAB + TB - 1) // TB
QROWS = NTB * (TB // 4)


def _tc_relayout(tableT):
    def body(in_ref, out_ref):
        x = in_ref[...]
        for k in range(TB // 4096):
            z = jnp.concatenate(
                [x[:, (4 * k + s) * 1024:(4 * k + s + 1) * 1024]
                 for s in range(4)], axis=0)
            out_ref[k * 1024:(k + 1) * 1024, :] = z.T

    return pl.pallas_call(
        body,
        grid=(NTB,),
        in_specs=[pl.BlockSpec((D, TB), lambda j: (0, j))],
        out_specs=pl.BlockSpec((TB // 4, 4 * D), lambda j: (j, 0)),
        out_shape=jax.ShapeDtypeStruct((QROWS, 4 * D), jnp.float32),
    )(tableT)


def _sc_bag(qidx, off, table4):
    mesh = plsc.VectorSubcoreMesh(core_axis_name="c", subcore_axis_name="s")

    @functools.partial(
        pl.kernel,
        mesh=mesh,
        out_type=jax.ShapeDtypeStruct((B // 4, 128), jnp.float32),
        scratch_types=[
            pltpu.VMEM((IDX_PER_W,), jnp.int32),
            pltpu.VMEM((IDX_PER_W,), jnp.int32),
            pltpu.VMEM((CH_ROWS, 128), jnp.float32),
            pltpu.VMEM((B_PER_W // 4, 128), jnp.float32),
            pltpu.SemaphoreType.DMA,
        ],
        compiler_params=pltpu.CompilerParams(needs_layout_passes=False),
    )
    def k(qidx_ref, off_ref, table_ref, bag_ref, qidx_v, off_v, buf, bag_v, sem):
        iota = lax.iota(jnp.int32, 16)
        wid = lax.axis_index("s") * _NC + lax.axis_index("c")
        base = wid * IDX_PER_W
        pltpu.sync_copy(qidx_ref.at[pl.ds(base, IDX_PER_W)], qidx_v)
        pltpu.sync_copy(off_ref.at[pl.ds(base, IDX_PER_W)], off_v)

        def chunk_body(c, carry):
            cbase = c * CH_ROWS
            cps = []
            for s, n in ((0, 128), (128, 128), (256, 64)):
                cps.append(pltpu.async_copy(
                    table_ref.at[qidx_v.at[pl.ds(cbase + s, n)]],
                    buf.at[pl.ds(s, n)],
                    sem))
            for cp in cps:
                cp.wait()
            for pair in range(CH_ELEMS // 2):
                for j in range(2):
                    e_lane = ((pair * 2 + j) % 4) * 32
                    e_row = c * (CH_ELEMS // 4) + pair // 2
                    for l in range(HIST):
                        r = (pair * 2 + j) * HIST + l
                        off_splat = plsc.load_gather(
                            off_v, [jnp.full((16,), cbase + r, jnp.int32)])
                        rowvec = jnp.full((16,), r, jnp.int32)
                        lanes0 = off_splat + iota
                        v0 = plsc.load_gather(buf, [rowvec, lanes0])
                        v1 = plsc.load_gather(buf, [rowvec, lanes0 + 16])
                        if l == 0:
                            acc0, acc1 = v0, v1
                        else:
                            acc0 = acc0 + v0
                            acc1 = acc1 + v1
                    bag_v[e_row, pl.ds(e_lane, 16)] = acc0
                    bag_v[e_row, pl.ds(e_lane + 16, 16)] = acc1
            return carry

        lax.fori_loop(0, NCHUNK, chunk_body, 0)
        pltpu.sync_copy(
            bag_v, bag_ref.at[pl.ds(wid * (B_PER_W // 4), B_PER_W // 4)])

    return k(qidx, off, table4)


def _tc_project_t(bag, W, b_scaled):
    BN = 1024

    def mm(w_ref, bag_ref, b_ref, out_ref):
        acc = lax.dot_general(
            w_ref[...], bag_ref[...],
            (((1,), (1,)), ((), ())),
            preferred_element_type=jnp.float32)
        out_ref[...] = acc + b_ref[...]

    return pl.pallas_call(
        mm,
        grid=(B // BN,),
        in_specs=[
            pl.BlockSpec((C, D), lambda j: (0, 0)),
            pl.BlockSpec((BN, D), lambda j: (j, 0)),
            pl.BlockSpec((C, 1), lambda j: (0, 0)),
        ],
        out_specs=pl.BlockSpec((C, BN), lambda j: (0, j)),
        out_shape=jax.ShapeDtypeStruct((C, B), jnp.float32),
    )(W, bag, b_scaled)


def kernel(inputs, embed_table, W, b):
    idx_flat = inputs.reshape(-1).astype(jnp.int32)
    qidx = ((idx_flat >> 12) << 10) | (idx_flat & 1023)
    off = ((idx_flat >> 10) & 3) << 5
    table4 = _tc_relayout(embed_table.T)
    bag = _sc_bag(qidx, off, table4).reshape(B, D)
    b_scaled = (b * jnp.float32(HIST)).reshape(C, 1)
    return _tc_project_t(bag, W, b_scaled).T


# speedup vs baseline: 2.4179x; 1.0079x over previous
import functools

import jax
import jax.numpy as jnp
from jax import lax
from jax.experimental import pallas as pl
from jax.experimental.pallas import tpu as pltpu
from jax.experimental.pallas import tpu_sc as plsc

VOCAB = 1000000
D = 32
B = 4096
HIST = 20
C = 1000

_info = plsc.get_sparse_core_info()
_NC, _NS, _L = _info.num_cores, _info.num_subcores, _info.num_lanes
NW = _NC * _NS
B_PER_W = B // NW
IDX_PER_W = B_PER_W * HIST
CH_ELEMS = 16
CH_ROWS = CH_ELEMS * HIST
NCHUNK = B_PER_W // CH_ELEMS
TB = 65536
NTB = (VOCAB + TB - 1) // TB
QROWS = NTB * (TB // 4)


def _tc_relayout(tableT):
    def body(in_ref, out_ref):
        x = in_ref[...]
        for k in range(TB // 4096):
            z = jnp.concatenate(
                [x[:, (4 * k + s) * 1024:(4 * k + s + 1) * 1024]
                 for s in range(4)], axis=0)
            out_ref[k * 1024:(k + 1) * 1024, :] = z.T

    return pl.pallas_call(
        body,
        grid=(NTB,),
        in_specs=[pl.BlockSpec((D, TB), lambda j: (0, j))],
        out_specs=pl.BlockSpec((TB // 4, 4 * D), lambda j: (j, 0)),
        out_shape=jax.ShapeDtypeStruct((QROWS, 4 * D), jnp.float32),
    )(tableT)


def _sc_bag(qidx, off, table4):
    mesh = plsc.VectorSubcoreMesh(core_axis_name="c", subcore_axis_name="s")

    @functools.partial(
        pl.kernel,
        mesh=mesh,
        out_type=jax.ShapeDtypeStruct((B // 4, 128), jnp.float32),
        scratch_types=[
            pltpu.VMEM((IDX_PER_W,), jnp.int32),
            pltpu.VMEM((IDX_PER_W,), jnp.int32),
            pltpu.VMEM((CH_ROWS, 128), jnp.float32),
            pltpu.VMEM((B_PER_W // 4, 128), jnp.float32),
            pltpu.SemaphoreType.DMA,
        ],
        compiler_params=pltpu.CompilerParams(needs_layout_passes=False),
    )
    def k(qidx_ref, off_ref, table_ref, bag_ref, qidx_v, off_v, buf, bag_v, sem):
        iota = lax.iota(jnp.int32, 16)
        wid = lax.axis_index("s") * _NC + lax.axis_index("c")
        base = wid * IDX_PER_W
        pltpu.sync_copy(qidx_ref.at[pl.ds(base, IDX_PER_W)], qidx_v)
        pltpu.sync_copy(off_ref.at[pl.ds(base, IDX_PER_W)], off_v)

        def chunk_body(c, carry):
            cbase = c * CH_ROWS
            cps = []
            for s, n in ((0, 128), (128, 128), (256, 64)):
                cps.append(pltpu.async_copy(
                    table_ref.at[qidx_v.at[pl.ds(cbase + s, n)]],
                    buf.at[pl.ds(s, n)],
                    sem))
            for cp in cps:
                cp.wait()
            for pair in range(CH_ELEMS // 2):
                for j in range(2):
                    e_lane = ((pair * 2 + j) % 4) * 32
                    e_row = c * (CH_ELEMS // 4) + pair // 2
                    for l in range(HIST):
                        r = (pair * 2 + j) * HIST + l
                        off_splat = plsc.load_gather(
                            off_v, [jnp.full((16,), cbase + r, jnp.int32)])
                        rowvec = jnp.full((16,), r, jnp.int32)
                        lanes0 = off_splat + iota
                        v0 = plsc.load_gather(buf, [rowvec, lanes0])
                        v1 = plsc.load_gather(buf, [rowvec, lanes0 + 16])
                        if l == 0:
                            acc0, acc1 = v0, v1
                        else:
                            acc0 = acc0 + v0
                            acc1 = acc1 + v1
                    bag_v[e_row, pl.ds(e_lane, 16)] = acc0
                    bag_v[e_row, pl.ds(e_lane + 16, 16)] = acc1
            return carry

        lax.fori_loop(0, NCHUNK, chunk_body, 0)
        pltpu.sync_copy(
            bag_v, bag_ref.at[pl.ds(wid * (B_PER_W // 4), B_PER_W // 4)])

    return k(qidx, off, table4)


def _tc_project_t(bag, W, b_scaled):
    BN = 1024

    def mm(w_ref, bag_ref, b_ref, out_ref):
        acc = lax.dot_general(
            w_ref[...], bag_ref[...],
            (((1,), (1,)), ((), ())),
            preferred_element_type=jnp.float32)
        out_ref[...] = acc + b_ref[...]

    return pl.pallas_call(
        mm,
        grid=(B // BN,),
        in_specs=[
            pl.BlockSpec((C, D), lambda j: (0, 0)),
            pl.BlockSpec((BN, D), lambda j: (j, 0)),
            pl.BlockSpec((C, 1), lambda j: (0, 0)),
        ],
        out_specs=pl.BlockSpec((C, BN), lambda j: (0, j)),
        out_shape=jax.ShapeDtypeStruct((C, B), jnp.float32),
    )(W, bag, b_scaled)


def kernel(inputs, embed_table, W, b):
    idx_flat = inputs.reshape(-1).astype(jnp.int32)
    qidx = ((idx_flat >> 12) << 10) | (idx_flat & 1023)
    off = ((idx_flat >> 10) & 3) << 5
    table4 = _tc_relayout(embed_table.T)
    bag = _sc_bag(qidx, off, table4).reshape(B, D)
    b_scaled = (b * jnp.float32(HIST)).reshape(C, 1)
    return _tc_project_t(bag, W, b_scaled).T
